# baseline (speedup 1.0000x reference)
import functools

import jax
import jax.numpy as jnp
from jax import lax
from jax.experimental import pallas as pl
from jax.experimental.pallas import tpu as pltpu
from jax.experimental.pallas import tpu_sc as plsc

N = 10000
E = 320000
LANES = 16
K = 80
N_SUBCORES = 16
N_CORES = 2
BN_SCALE = float(1.0 / (257.0 ** 0.5))
NEG_SLOPE = 0.01


def _make_sc_edge_layer(n, e, dh, feature_split):
    assert e % K == 0 and dh % LANES == 0
    n_chunks = e // K
    if feature_split:
        core_chunks = n_chunks
    else:
        assert n_chunks % N_CORES == 0
        core_chunks = n_chunks // N_CORES
    chunks_per_tile = core_chunks // N_SUBCORES
    assert core_chunks % N_SUBCORES == 0
    nf = dh // LANES
    ZR = K
    RPT = (n // N_SUBCORES) // 8 * 8
    LAST = n - (N_SUBCORES - 1) * RPT
    assert RPT % 8 == 0 and LAST % 8 == 0 and ZR % 8 == 0
    RPT_FULL, RPT_TAIL = RPT // ZR, RPT % ZR
    LAST_FULL, LAST_TAIL = LAST // ZR, LAST % ZR
    pipe_pairs = chunks_per_tile // 2 * 2
    odd_chunk = chunks_per_tile - pipe_pairs

    mesh = plsc.VectorSubcoreMesh(core_axis_name="c", subcore_axis_name="s")

    @functools.partial(
        pl.kernel,
        out_type=jax.ShapeDtypeStruct((N_CORES * n, dh), jnp.float32),
        mesh=mesh,
        scratch_types=[
            pltpu.VMEM_SHARED((n, dh), jnp.float32),
            pltpu.VMEM((2, 4, K), jnp.int32),
            pltpu.VMEM((2, K, dh), jnp.float32),
            pltpu.VMEM((2, K, dh), jnp.float32),
            pltpu.SemaphoreType.DMA((2,)),
            pltpu.SemaphoreType.DMA((2,)),
            pltpu.SemaphoreType.DMA((2,)),
        ],
    )
    def sc_kernel(xs_hbm, es_hbm, ei_hbm, aggs_hbm,
                  acc, idxb, xbuf, ebuf, gsem, esem, ssem):
        c = lax.axis_index("c")
        s = lax.axis_index("s")
        cn = c * n
        if feature_split:
            chunk0, es_off = 0, c * e
        else:
            chunk0, es_off = c * core_chunks, 0

        @pl.loop(0, ZR)
        def _zero_rows(r):
            for f in range(nf):
                ebuf[0, r, pl.ds(f * LANES, LANES)] = jnp.zeros((LANES,),
                                                                jnp.float32)

        row0 = pl.multiple_of(s * RPT, 8)

        def for_each_span(fn):
            for kk in range(RPT_FULL):
                fn(pl.multiple_of(row0 + kk * ZR, 8), ZR)

            @pl.when(s < N_SUBCORES - 1)
            def _tail_std():
                if RPT_TAIL:
                    fn(pl.multiple_of(row0 + RPT_FULL * ZR, 8), RPT_TAIL)

            @pl.when(s == N_SUBCORES - 1)
            def _tail_last():
                for kk in range(RPT_FULL, LAST_FULL):
                    fn(pl.multiple_of(row0 + kk * ZR, 8), ZR)
                if LAST_TAIL:
                    fn(pl.multiple_of(row0 + LAST_FULL * ZR, 8), LAST_TAIL)

        def chunk_of(j):
            return chunk0 + j * N_SUBCORES + s

        gather_row = c if feature_split else 0

        def wait_scatter(p):
            pltpu.make_async_copy(ebuf.at[p], acc.at[idxb.at[p, 2]],
                                  ssem.at[p]).wait()

        def prefetch(chunk_id, p, drain_scatter):
            if drain_scatter:
                wait_scatter(p)
            base = pl.multiple_of(chunk_id * K, 8)
            pltpu.sync_copy(ei_hbm.at[chunk_id], idxb.at[p])
            pltpu.async_copy(xs_hbm.at[idxb.at[p, gather_row]], xbuf.at[p],
                             gsem.at[p])
            pltpu.async_copy(es_hbm.at[pl.ds(es_off + base, K)], ebuf.at[p],
                             esem.at[p])

        def finish(chunk_id, p):
            del chunk_id
            pltpu.make_async_copy(xs_hbm.at[idxb.at[p, gather_row]], xbuf.at[p],
                                  gsem.at[p]).wait()
            pltpu.make_async_copy(es_hbm.at[pl.ds(0, K)], ebuf.at[p],
                                  esem.at[p]).wait()

            @pl.loop(0, K, step=4)
            def _relu_rows(r):
                for rr in range(4):
                    for f in range(nf):
                        sl = pl.ds(f * LANES, LANES)
                        ebuf[p, r + rr, sl] = jnp.maximum(
                            xbuf[p, r + rr, sl] + ebuf[p, r + rr, sl], 0.0)

            pltpu.async_copy(ebuf.at[p], acc.at[idxb.at[p, 2]], ssem.at[p],
                             add=True)

        for_each_span(lambda r0, sz: pltpu.sync_copy(
            ebuf.at[0, pl.ds(0, sz)], acc.at[pl.ds(r0, sz)]))
        prefetch(chunk_of(0), 0, drain_scatter=False)
        prefetch(chunk_of(1), 1, drain_scatter=False)
        plsc.subcore_barrier()

        @pl.loop(0, pipe_pairs, step=2)
        def _chunks(j):
            finish(chunk_of(j), 0)

            @pl.when(j + 2 < chunks_per_tile)
            def _pf0():
                prefetch(chunk_of(j + 2), 0, drain_scatter=True)

            finish(chunk_of(j + 1), 1)

            @pl.when(j + 3 < chunks_per_tile)
            def _pf1():
                prefetch(chunk_of(j + 3), 1, drain_scatter=True)

        for _ in range(odd_chunk):
            finish(chunk_of(chunks_per_tile - 1), 0)

        wait_scatter(0)
        wait_scatter(1)
        plsc.subcore_barrier()

        def copy_out(r0, sz):
            pltpu.sync_copy(acc.at[pl.ds(r0, sz)],
                            aggs_hbm.at[pl.ds(pl.multiple_of(cn + r0, 8), sz)])

        for_each_span(copy_out)

    return sc_kernel


def _edge_lin1(edge_attr, We1, be1):
    BE = 8000

    def body(ea, w1, b1, o1):
        o1[...] = lax.dot_general(ea[...], w1[...], (((1,), (1,)), ((), ())),
                                  preferred_element_type=jnp.float32) + b1[...]

    return pl.pallas_call(
        body,
        grid=(E // BE,),
        in_specs=[
            pl.BlockSpec((BE, 4), lambda i: (i, 0)),
            pl.BlockSpec((128, 4), lambda i: (0, 0)),
            pl.BlockSpec((1, 128), lambda i: (0, 0)),
        ],
        out_specs=[pl.BlockSpec((BE, 128), lambda i: (i, 0))],
        out_shape=[jax.ShapeDtypeStruct((E, 128), jnp.float32)],
    )(edge_attr, We1, be1.reshape(1, 128))[0]


def _edge_lin23(edge_attr, We2, be2, We3, be3):
    BE = 8000
    grid = (E // BE, N_CORES)

    def body(ea, w2, b2, w3, b3, o2, o3):
        ea_v = ea[...]
        for w, b, o in ((w2, b2, o2), (w3, b3, o3)):
            o[0] = lax.dot_general(ea_v, w[0], (((1,), (1,)), ((), ())),
                                   preferred_element_type=jnp.float32) + b[0]

    def wspec():
        return pl.BlockSpec((1, 128, 4), lambda i, c: (c, 0, 0))

    def bspec():
        return pl.BlockSpec((1, 1, 128), lambda i, c: (c, 0, 0))

    def ospec():
        return pl.BlockSpec((1, BE, 128), lambda i, c: (c, i, 0))

    return pl.pallas_call(
        body,
        grid=grid,
        in_specs=[
            pl.BlockSpec((BE, 4), lambda i, c: (i, 0)),
            wspec(), bspec(), wspec(), bspec(),
        ],
        out_specs=[ospec(), ospec()],
        out_shape=[
            jax.ShapeDtypeStruct((N_CORES, E, 128), jnp.float32),
            jax.ShapeDtypeStruct((N_CORES, E, 128), jnp.float32),
        ],
    )(edge_attr,
      We2.reshape(N_CORES, 128, 4), be2.reshape(N_CORES, 1, 128),
      We3.reshape(N_CORES, 128, 4), be3.reshape(N_CORES, 1, 128))


def _node_mlp(x_prev, aggs, Wn, bn, g, b, d_in, concat_agg):
    BN = 1000

    def body(xp, ag, w, bn_r, g_r, b_r, h_ref, hs_ref):
        if concat_agg:
            inp = jnp.concatenate([ag[0], ag[1]], axis=-1) + xp[...]
        else:
            inp = ag[0] + ag[1] + xp[...]
        h = lax.dot_general(inp, w[...], (((1,), (1,)), ((), ())),
                            preferred_element_type=jnp.float32)
        scale = g_r[...] * BN_SCALE
        h = (h + bn_r[...]) * scale + b_r[...]
        h = jnp.where(h >= 0.0, h, NEG_SLOPE * h)
        h_ref[...] = h
        hs_ref[0] = h[:, :128]
        hs_ref[1] = h[:, 128:]

    return pl.pallas_call(
        body,
        grid=(N // BN,),
        in_specs=[
            pl.BlockSpec((BN, d_in), lambda i: (i, 0)),
            pl.BlockSpec((2, BN, 128), lambda i: (0, i, 0)),
            pl.BlockSpec((256, d_in), lambda i: (0, 0)),
            pl.BlockSpec((1, 256), lambda i: (0, 0)),
            pl.BlockSpec((1, 256), lambda i: (0, 0)),
            pl.BlockSpec((1, 256), lambda i: (0, 0)),
        ],
        out_specs=[
            pl.BlockSpec((BN, 256), lambda i: (i, 0)),
            pl.BlockSpec((2, BN, 128), lambda i: (0, i, 0)),
        ],
        out_shape=[
            jax.ShapeDtypeStruct((N, 256), jnp.float32),
            jax.ShapeDtypeStruct((2, N, 128), jnp.float32),
        ],
    )(x_prev, aggs, Wn, bn.reshape(1, 256), g.reshape(1, 256), b.reshape(1, 256))


def _head(h1, h2, h3, Wl1, bl1, Wl2, bl2):
    BN = 1000

    def body(r1, r2, r3, w1, b1, w2, b2, y_ref, p_ref):
        hcat = jnp.concatenate([r1[...], r2[...], r3[...]], axis=-1)
        y1 = lax.dot_general(hcat, w1[...], (((1,), (1,)), ((), ())),
                             preferred_element_type=jnp.float32) + b1[...]
        y1 = jnp.maximum(y1, 0.0)
        y2 = lax.dot_general(y1, w2[...], (((1,), (1,)), ((), ())),
                             preferred_element_type=jnp.float32) + b2[...]
        y_ref[...] = y2
        m = jnp.max(y2, axis=-1, keepdims=True)
        ex = jnp.exp(y2 - m)
        p_ref[...] = ex / jnp.sum(ex, axis=-1, keepdims=True)

    return pl.pallas_call(
        body,
        grid=(N // BN,),
        in_specs=[
            pl.BlockSpec((BN, 256), lambda i: (i, 0)),
            pl.BlockSpec((BN, 256), lambda i: (i, 0)),
            pl.BlockSpec((BN, 256), lambda i: (i, 0)),
            pl.BlockSpec((768, 768), lambda i: (0, 0)),
            pl.BlockSpec((1, 768), lambda i: (0, 0)),
            pl.BlockSpec((64, 768), lambda i: (0, 0)),
            pl.BlockSpec((1, 64), lambda i: (0, 0)),
        ],
        out_specs=[
            pl.BlockSpec((BN, 64), lambda i: (i, 0)),
            pl.BlockSpec((BN, 64), lambda i: (i, 0)),
        ],
        out_shape=[
            jax.ShapeDtypeStruct((N, 64), jnp.float32),
            jax.ShapeDtypeStruct((N, 64), jnp.float32),
        ],
    )(h1, h2, h3, Wl1, bl1.reshape(1, 768), Wl2, bl2.reshape(1, 64))


_sc_layer_edge_split = _make_sc_edge_layer(N, E, 128, feature_split=False)
_sc_layer_feat_split = _make_sc_edge_layer(N, E, 128, feature_split=True)


def kernel(x, edge_index, edge_attr,
           We1, be1, Wn1, bn1, g1, b1,
           We2, be2, Wn2, bn2, g2, b2,
           We3, be3, Wn3, bn3, g3, b3,
           Wl1, bl1, Wl2, bl2):
    es1 = _edge_lin1(edge_attr, We1, be1)
    s_chunks = edge_index[0].reshape(E // K, K)
    d_chunks = edge_index[1].reshape(E // K, K)
    ei_packed = jnp.stack([s_chunks, s_chunks + N, d_chunks, d_chunks], axis=1)

    aggs1 = _sc_layer_edge_split(x, es1, ei_packed)
    es2, es3 = _edge_lin23(edge_attr, We2, be2, We3, be3)
    es2 = es2.reshape(N_CORES * E, 128)
    es3 = es3.reshape(N_CORES * E, 128)
    h1, h1s = _node_mlp(x, aggs1.reshape(2, N, 128), Wn1, bn1, g1, b1, 128,
                        concat_agg=False)

    aggs2 = _sc_layer_feat_split(h1s.reshape(N_CORES * N, 128), es2, ei_packed)
    h2, h2s = _node_mlp(h1, aggs2.reshape(2, N, 128), Wn2, bn2, g2, b2, 256,
                        concat_agg=True)

    aggs3 = _sc_layer_feat_split(h2s.reshape(N_CORES * N, 128), es3, ei_packed)
    h3, _ = _node_mlp(h2, aggs3.reshape(2, N, 128), Wn3, bn3, g3, b3, 256,
                      concat_agg=True)

    return _head(h1, h2, h3, Wl1, bl1, Wl2, bl2)


# speedup vs baseline: 1.0152x; 1.0152x over previous
import functools

import jax
import jax.numpy as jnp
from jax import lax
from jax.experimental import pallas as pl
from jax.experimental.pallas import tpu as pltpu
from jax.experimental.pallas import tpu_sc as plsc

N = 10000
E = 320000
LANES = 16
K = 80
N_SUBCORES = 16
N_CORES = 2
BN_SCALE = float(1.0 / (257.0 ** 0.5))
NEG_SLOPE = 0.01


def _make_sc_edge_layer(n, e, dh, feature_split):
    assert e % K == 0 and dh % LANES == 0
    n_chunks = e // K
    if feature_split:
        core_chunks = n_chunks
    else:
        assert n_chunks % N_CORES == 0
        core_chunks = n_chunks // N_CORES
    chunks_per_tile = core_chunks // N_SUBCORES
    assert core_chunks % N_SUBCORES == 0
    nf = dh // LANES
    ZR = K
    RPT = (n // N_SUBCORES) // 8 * 8
    LAST = n - (N_SUBCORES - 1) * RPT
    assert RPT % 8 == 0 and LAST % 8 == 0 and ZR % 8 == 0
    RPT_FULL, RPT_TAIL = RPT // ZR, RPT % ZR
    LAST_FULL, LAST_TAIL = LAST // ZR, LAST % ZR
    pipe_pairs = chunks_per_tile // 2 * 2
    odd_chunk = chunks_per_tile - pipe_pairs

    mesh = plsc.VectorSubcoreMesh(core_axis_name="c", subcore_axis_name="s")

    @functools.partial(
        pl.kernel,
        out_type=jax.ShapeDtypeStruct((N_CORES * n, dh), jnp.float32),
        mesh=mesh,
        scratch_types=[
            pltpu.VMEM_SHARED((n, dh), jnp.float32),
            pltpu.VMEM((2, 2, K), jnp.int32),
            pltpu.VMEM((2, K), jnp.int32),
            pltpu.VMEM((2, K, dh), jnp.float32),
            pltpu.VMEM((2, K, dh), jnp.float32),
            pltpu.SemaphoreType.DMA((2,)),
            pltpu.SemaphoreType.DMA((2,)),
            pltpu.SemaphoreType.DMA((2,)),
        ],
    )
    def sc_kernel(xs_hbm, es_hbm, ei_hbm, aggs_hbm,
                  acc, idxb, gidxb, xbuf, ebuf, gsem, esem, ssem):
        c = lax.axis_index("c")
        s = lax.axis_index("s")
        cn = c * n
        if feature_split:
            chunk0, idx_off, es_off = 0, cn, c * e
        else:
            chunk0, idx_off, es_off = c * core_chunks, 0, 0

        @pl.loop(0, ZR)
        def _zero_rows(r):
            for f in range(nf):
                ebuf[0, r, pl.ds(f * LANES, LANES)] = jnp.zeros((LANES,),
                                                                jnp.float32)

        row0 = pl.multiple_of(s * RPT, 8)

        def for_each_span(fn):
            for kk in range(RPT_FULL):
                fn(pl.multiple_of(row0 + kk * ZR, 8), ZR)

            @pl.when(s < N_SUBCORES - 1)
            def _tail_std():
                if RPT_TAIL:
                    fn(pl.multiple_of(row0 + RPT_FULL * ZR, 8), RPT_TAIL)

            @pl.when(s == N_SUBCORES - 1)
            def _tail_last():
                for kk in range(RPT_FULL, LAST_FULL):
                    fn(pl.multiple_of(row0 + kk * ZR, 8), ZR)
                if LAST_TAIL:
                    fn(pl.multiple_of(row0 + LAST_FULL * ZR, 8), LAST_TAIL)

        def chunk_of(j):
            return chunk0 + j * N_SUBCORES + s

        def wait_scatter(p):
            pltpu.make_async_copy(ebuf.at[p], acc.at[idxb.at[p, 1]],
                                  ssem.at[p]).wait()

        def prefetch(chunk_id, p, drain_scatter):
            if drain_scatter:
                wait_scatter(p)
            base = pl.multiple_of(chunk_id * K, 8)
            pltpu.sync_copy(ei_hbm.at[chunk_id], idxb.at[p])
            pltpu.async_copy(es_hbm.at[pl.ds(es_off + base, K)], ebuf.at[p],
                             esem.at[p])

            @pl.loop(0, K // LANES)
            def _mk_idx(i):
                sl = pl.ds(i * LANES, LANES)
                gidxb[p, sl] = idxb[p, 0, sl] + idx_off

            pltpu.async_copy(xs_hbm.at[gidxb.at[p]], xbuf.at[p], gsem.at[p])

        def finish(chunk_id, p):
            del chunk_id
            pltpu.make_async_copy(xs_hbm.at[gidxb.at[p]], xbuf.at[p],
                                  gsem.at[p]).wait()
            pltpu.make_async_copy(es_hbm.at[pl.ds(0, K)], ebuf.at[p],
                                  esem.at[p]).wait()

            @pl.loop(0, K, step=4)
            def _relu_rows(r):
                for rr in range(4):
                    for f in range(nf):
                        sl = pl.ds(f * LANES, LANES)
                        ebuf[p, r + rr, sl] = jnp.maximum(
                            xbuf[p, r + rr, sl] + ebuf[p, r + rr, sl], 0.0)

            pltpu.async_copy(ebuf.at[p], acc.at[idxb.at[p, 1]], ssem.at[p],
                             add=True)

        for_each_span(lambda r0, sz: pltpu.sync_copy(
            ebuf.at[0, pl.ds(0, sz)], acc.at[pl.ds(r0, sz)]))
        prefetch(chunk_of(0), 0, drain_scatter=False)
        prefetch(chunk_of(1), 1, drain_scatter=False)
        plsc.subcore_barrier()

        @pl.loop(0, pipe_pairs, step=2)
        def _chunks(j):
            finish(chunk_of(j), 0)

            @pl.when(j + 2 < chunks_per_tile)
            def _pf0():
                prefetch(chunk_of(j + 2), 0, drain_scatter=True)

            finish(chunk_of(j + 1), 1)

            @pl.when(j + 3 < chunks_per_tile)
            def _pf1():
                prefetch(chunk_of(j + 3), 1, drain_scatter=True)

        for _ in range(odd_chunk):
            finish(chunk_of(chunks_per_tile - 1), 0)

        wait_scatter(0)
        wait_scatter(1)
        plsc.subcore_barrier()

        def copy_out(r0, sz):
            pltpu.sync_copy(acc.at[pl.ds(r0, sz)],
                            aggs_hbm.at[pl.ds(pl.multiple_of(cn + r0, 8), sz)])

        for_each_span(copy_out)

    return sc_kernel


def _edge_lin1(edge_attr, We1, be1):
    BE = 8000

    def body(ea, w1, b1, o1):
        o1[...] = lax.dot_general(ea[...], w1[...], (((1,), (1,)), ((), ())),
                                  preferred_element_type=jnp.float32) + b1[...]

    return pl.pallas_call(
        body,
        grid=(E // BE,),
        in_specs=[
            pl.BlockSpec((BE, 4), lambda i: (i, 0)),
            pl.BlockSpec((128, 4), lambda i: (0, 0)),
            pl.BlockSpec((1, 128), lambda i: (0, 0)),
        ],
        out_specs=[pl.BlockSpec((BE, 128), lambda i: (i, 0))],
        out_shape=[jax.ShapeDtypeStruct((E, 128), jnp.float32)],
    )(edge_attr, We1, be1.reshape(1, 128))[0]


def _edge_lin23(edge_attr, We2, be2, We3, be3):
    BE = 8000
    grid = (E // BE, N_CORES)

    def body(ea, w2, b2, w3, b3, o2, o3):
        ea_v = ea[...]
        for w, b, o in ((w2, b2, o2), (w3, b3, o3)):
            o[0] = lax.dot_general(ea_v, w[0], (((1,), (1,)), ((), ())),
                                   preferred_element_type=jnp.float32) + b[0]

    def wspec():
        return pl.BlockSpec((1, 128, 4), lambda i, c: (c, 0, 0))

    def bspec():
        return pl.BlockSpec((1, 1, 128), lambda i, c: (c, 0, 0))

    def ospec():
        return pl.BlockSpec((1, BE, 128), lambda i, c: (c, i, 0))

    return pl.pallas_call(
        body,
        grid=grid,
        in_specs=[
            pl.BlockSpec((BE, 4), lambda i, c: (i, 0)),
            wspec(), bspec(), wspec(), bspec(),
        ],
        out_specs=[ospec(), ospec()],
        out_shape=[
            jax.ShapeDtypeStruct((N_CORES, E, 128), jnp.float32),
            jax.ShapeDtypeStruct((N_CORES, E, 128), jnp.float32),
        ],
    )(edge_attr,
      We2.reshape(N_CORES, 128, 4), be2.reshape(N_CORES, 1, 128),
      We3.reshape(N_CORES, 128, 4), be3.reshape(N_CORES, 1, 128))


def _node_mlp(x_prev, aggs, Wn, bn, g, b, d_in, concat_agg):
    BN = 1000

    def body(xp, ag, w, bn_r, g_r, b_r, h_ref, hs_ref):
        if concat_agg:
            inp = jnp.concatenate([ag[0], ag[1]], axis=-1) + xp[...]
        else:
            inp = ag[0] + ag[1] + xp[...]
        h = lax.dot_general(inp, w[...], (((1,), (1,)), ((), ())),
                            preferred_element_type=jnp.float32)
        scale = g_r[...] * BN_SCALE
        h = (h + bn_r[...]) * scale + b_r[...]
        h = jnp.where(h >= 0.0, h, NEG_SLOPE * h)
        h_ref[...] = h
        hs_ref[0] = h[:, :128]
        hs_ref[1] = h[:, 128:]

    return pl.pallas_call(
        body,
        grid=(N // BN,),
        in_specs=[
            pl.BlockSpec((BN, d_in), lambda i: (i, 0)),
            pl.BlockSpec((2, BN, 128), lambda i: (0, i, 0)),
            pl.BlockSpec((256, d_in), lambda i: (0, 0)),
            pl.BlockSpec((1, 256), lambda i: (0, 0)),
            pl.BlockSpec((1, 256), lambda i: (0, 0)),
            pl.BlockSpec((1, 256), lambda i: (0, 0)),
        ],
        out_specs=[
            pl.BlockSpec((BN, 256), lambda i: (i, 0)),
            pl.BlockSpec((2, BN, 128), lambda i: (0, i, 0)),
        ],
        out_shape=[
            jax.ShapeDtypeStruct((N, 256), jnp.float32),
            jax.ShapeDtypeStruct((2, N, 128), jnp.float32),
        ],
    )(x_prev, aggs, Wn, bn.reshape(1, 256), g.reshape(1, 256), b.reshape(1, 256))


def _head(h1, h2, h3, Wl1, bl1, Wl2, bl2):
    BN = 1000

    def body(r1, r2, r3, w1, b1, w2, b2, y_ref, p_ref):
        hcat = jnp.concatenate([r1[...], r2[...], r3[...]], axis=-1)
        y1 = lax.dot_general(hcat, w1[...], (((1,), (1,)), ((), ())),
                             preferred_element_type=jnp.float32) + b1[...]
        y1 = jnp.maximum(y1, 0.0)
        y2 = lax.dot_general(y1, w2[...], (((1,), (1,)), ((), ())),
                             preferred_element_type=jnp.float32) + b2[...]
        y_ref[...] = y2
        m = jnp.max(y2, axis=-1, keepdims=True)
        ex = jnp.exp(y2 - m)
        p_ref[...] = ex / jnp.sum(ex, axis=-1, keepdims=True)

    return pl.pallas_call(
        body,
        grid=(N // BN,),
        in_specs=[
            pl.BlockSpec((BN, 256), lambda i: (i, 0)),
            pl.BlockSpec((BN, 256), lambda i: (i, 0)),
            pl.BlockSpec((BN, 256), lambda i: (i, 0)),
            pl.BlockSpec((768, 768), lambda i: (0, 0)),
            pl.BlockSpec((1, 768), lambda i: (0, 0)),
            pl.BlockSpec((64, 768), lambda i: (0, 0)),
            pl.BlockSpec((1, 64), lambda i: (0, 0)),
        ],
        out_specs=[
            pl.BlockSpec((BN, 64), lambda i: (i, 0)),
            pl.BlockSpec((BN, 64), lambda i: (i, 0)),
        ],
        out_shape=[
            jax.ShapeDtypeStruct((N, 64), jnp.float32),
            jax.ShapeDtypeStruct((N, 64), jnp.float32),
        ],
    )(h1, h2, h3, Wl1, bl1.reshape(1, 768), Wl2, bl2.reshape(1, 64))


_sc_layer_edge_split = _make_sc_edge_layer(N, E, 128, feature_split=False)
_sc_layer_feat_split = _make_sc_edge_layer(N, E, 128, feature_split=True)


def kernel(x, edge_index, edge_attr,
           We1, be1, Wn1, bn1, g1, b1,
           We2, be2, Wn2, bn2, g2, b2,
           We3, be3, Wn3, bn3, g3, b3,
           Wl1, bl1, Wl2, bl2):
    es1 = _edge_lin1(edge_attr, We1, be1)
    ei_packed = edge_index.reshape(2, E // K, K).transpose(1, 0, 2)

    aggs1 = _sc_layer_edge_split(x, es1, ei_packed)
    es2, es3 = _edge_lin23(edge_attr, We2, be2, We3, be3)
    es2 = es2.reshape(N_CORES * E, 128)
    es3 = es3.reshape(N_CORES * E, 128)
    h1, h1s = _node_mlp(x, aggs1.reshape(2, N, 128), Wn1, bn1, g1, b1, 128,
                        concat_agg=False)

    aggs2 = _sc_layer_feat_split(h1s.reshape(N_CORES * N, 128), es2, ei_packed)
    h2, h2s = _node_mlp(h1, aggs2.reshape(2, N, 128), Wn2, bn2, g2, b2, 256,
                        concat_agg=True)

    aggs3 = _sc_layer_feat_split(h2s.reshape(N_CORES * N, 128), es3, ei_packed)
    h3, _ = _node_mlp(h2, aggs3.reshape(2, N, 128), Wn3, bn3, g3, b3, 256,
                      concat_agg=True)

    return _head(h1, h2, h3, Wl1, bl1, Wl2, bl2)
